# baseline (device time: 17163 ns/iter reference)
import jax
import jax.numpy as jnp
from jax import lax
from jax.experimental import pallas as pl
from jax.experimental.pallas import tpu as pltpu

N_DEV = 4
B, Sq, Skv, Dh = 2, 128, 128, 64
HL = 4
D_MODEL = 512
D_HEADS = HL * Dh
ROWS = B * Sq
BLK = 64


def _attn_block(q, k, v):
    s = lax.dot_general(q, k, (((1,), (1,)), ((), ())),
                        preferred_element_type=jnp.float32) * 0.125
    m = jnp.max(s, axis=-1, keepdims=True)
    w = jnp.exp(s - m)
    r = 1.0 / jnp.sum(w, axis=-1, keepdims=True)
    ctx = jnp.dot(w.astype(jnp.bfloat16), v,
                  preferred_element_type=jnp.float32)
    return ctx * r


def _body(x_ref, wq_ref, kf_ref, vf_ref, wo_ref, out_ref,
          ctx_ref, s1_ref, rp1_ref, rp2_ref, rpd_ref, send_sems, recv_sems):
    my_i = lax.axis_index("i")
    p1 = my_i ^ 1
    p2 = 3 - my_i
    pd = my_i ^ 2
    peers = (p1, p2, pd)

    barrier_sem = pltpu.get_barrier_semaphore()
    for nbr in peers:
        pl.semaphore_signal(
            barrier_sem, inc=1,
            device_id=(nbr,), device_id_type=pl.DeviceIdType.MESH,
        )
    pl.semaphore_wait(barrier_sem, 3)

    xb = x_ref[...].astype(jnp.bfloat16)
    wqb = wq_ref[...].astype(jnp.bfloat16)
    qf = jnp.dot(xb, wqb, preferred_element_type=jnp.float32)
    qfb = qf.astype(jnp.bfloat16)
    kb = kf_ref[...].astype(jnp.bfloat16)
    vb = vf_ref[...].astype(jnp.bfloat16)
    wob = wo_ref[...].astype(jnp.bfloat16)

    recv_refs = (rp1_ref, rp2_ref, rpd_ref)
    rdmas = []
    for b in range(B):
        rs = pl.ds(b * Sq, Sq)
        r0 = b * Sq
        for h in range(HL):
            cs = slice(h * Dh, (h + 1) * Dh)
            k = kb[r0:r0 + Skv, cs]
            v = vb[r0:r0 + Skv, cs]
            ctx_ref[0:BLK, cs] = _attn_block(
                qfb[r0:r0 + BLK, cs], k[0:BLK], v[0:BLK]
            ).astype(jnp.bfloat16)
            ctx_ref[BLK:Sq, cs] = _attn_block(
                qfb[r0 + BLK:r0 + Sq, cs], k, v
            ).astype(jnp.bfloat16)
        partial = jnp.dot(ctx_ref[...], wob,
                          preferred_element_type=jnp.float32)
        out_ref[rs, :] = partial
        s1_ref[b] = partial.astype(jnp.bfloat16)
        chunk = []
        for j, (peer, rref) in enumerate(zip(peers, recv_refs)):
            rdma = pltpu.make_async_remote_copy(
                src_ref=s1_ref.at[b],
                dst_ref=rref.at[b],
                send_sem=send_sems.at[j, b],
                recv_sem=recv_sems.at[j, b],
                device_id=(peer,),
                device_id_type=pl.DeviceIdType.MESH,
            )
            rdma.start()
            chunk.append(rdma)
        rdmas.append(chunk)

    for b in range(B):
        rs = pl.ds(b * Sq, Sq)
        for rdma in rdmas[b]:
            rdma.wait()
        out_ref[rs, :] = (
            out_ref[rs, :]
            + rp1_ref[b].astype(jnp.float32)
            + rp2_ref[b].astype(jnp.float32)
            + rpd_ref[b].astype(jnp.float32)
        )


def kernel(x, Wq, K_ext, V_ext, Wo):
    my_i = lax.axis_index("i")
    Kh = lax.dynamic_slice_in_dim(K_ext, my_i * HL, HL, axis=2)
    Vh = lax.dynamic_slice_in_dim(V_ext, my_i * HL, HL, axis=2)

    out = pl.pallas_call(
        _body,
        out_shape=jax.ShapeDtypeStruct((ROWS, D_MODEL), jnp.float32),
        in_specs=[pl.BlockSpec(memory_space=pltpu.VMEM)] * 5,
        out_specs=pl.BlockSpec(memory_space=pltpu.VMEM),
        scratch_shapes=[
            pltpu.VMEM((Sq, D_HEADS), jnp.bfloat16),
            pltpu.VMEM((B, Sq, D_MODEL), jnp.bfloat16),
            pltpu.VMEM((B, Sq, D_MODEL), jnp.bfloat16),
            pltpu.VMEM((B, Sq, D_MODEL), jnp.bfloat16),
            pltpu.VMEM((B, Sq, D_MODEL), jnp.bfloat16),
            pltpu.SemaphoreType.DMA((3, B)),
            pltpu.SemaphoreType.DMA((3, B)),
        ],
        compiler_params=pltpu.CompilerParams(collective_id=0),
    )(
        x.reshape(ROWS, D_MODEL),
        Wq,
        Kh.reshape(B * Skv, D_HEADS),
        Vh.reshape(B * Skv, D_HEADS),
        Wo,
    )
    return out.reshape(B, Sq, D_MODEL)


# device time: 16503 ns/iter; 1.0400x vs baseline; 1.0400x over previous
import jax
import jax.numpy as jnp
from jax import lax
from jax.experimental import pallas as pl
from jax.experimental.pallas import tpu as pltpu

N_DEV = 4
B, Sq, Skv, Dh = 2, 128, 128, 64
HL = 4
D_MODEL = 512
D_HEADS = HL * Dh
ROWS = B * Sq
BLK = 64
NCHUNK = ROWS // BLK


def _attn_block(q, k, v):
    s = lax.dot_general(q, k, (((1,), (1,)), ((), ())),
                        preferred_element_type=jnp.float32) * 0.125
    m = jnp.max(s, axis=-1, keepdims=True)
    w = jnp.exp(s - m)
    r = 1.0 / jnp.sum(w, axis=-1, keepdims=True)
    ctx = jnp.dot(w.astype(jnp.bfloat16), v,
                  preferred_element_type=jnp.float32)
    return ctx * r


def _body(x_ref, wq_ref, kf_ref, vf_ref, wo_ref, out_ref,
          ctx_ref, s1_ref, rp1_ref, rp2_ref, rpd_ref, send_sems, recv_sems):
    my_i = lax.axis_index("i")
    p1 = my_i ^ 1
    p2 = 3 - my_i
    pd = my_i ^ 2
    peers = (p1, p2, pd)

    barrier_sem = pltpu.get_barrier_semaphore()
    for nbr in peers:
        pl.semaphore_signal(
            barrier_sem, inc=1,
            device_id=(nbr,), device_id_type=pl.DeviceIdType.MESH,
        )

    xb = x_ref[...].astype(jnp.bfloat16)
    wqb = wq_ref[...].astype(jnp.bfloat16)
    qf = jnp.dot(xb, wqb, preferred_element_type=jnp.float32)
    qfb = qf.astype(jnp.bfloat16)
    kb = kf_ref[...].astype(jnp.bfloat16)
    vb = vf_ref[...].astype(jnp.bfloat16)
    wob = wo_ref[...].astype(jnp.bfloat16)

    recv_refs = (rp1_ref, rp2_ref, rpd_ref)
    rdmas = []
    for c in range(NCHUNK):
        b, half = divmod(c, 2)
        r0 = b * Sq + half * BLK
        k0 = b * Skv
        for h in range(HL):
            cs = slice(h * Dh, (h + 1) * Dh)
            q = qfb[r0:r0 + BLK, cs]
            if half == 0:
                k = kb[k0:k0 + BLK, cs]
                v = vb[k0:k0 + BLK, cs]
            else:
                k = kb[k0:k0 + Skv, cs]
                v = vb[k0:k0 + Skv, cs]
            ctx_ref[:, cs] = _attn_block(q, k, v).astype(jnp.bfloat16)
        partial = jnp.dot(ctx_ref[...], wob,
                          preferred_element_type=jnp.float32)
        out_ref[pl.ds(r0, BLK), :] = partial
        s1_ref[c] = partial.astype(jnp.bfloat16)
        if c == 0:
            pl.semaphore_wait(barrier_sem, 3)
        chunk = []
        for j, (peer, rref) in enumerate(zip(peers, recv_refs)):
            rdma = pltpu.make_async_remote_copy(
                src_ref=s1_ref.at[c],
                dst_ref=rref.at[c],
                send_sem=send_sems.at[j, c],
                recv_sem=recv_sems.at[j, c],
                device_id=(peer,),
                device_id_type=pl.DeviceIdType.MESH,
            )
            rdma.start()
            chunk.append(rdma)
        rdmas.append(chunk)

    for c in range(NCHUNK):
        b, half = divmod(c, 2)
        r0 = b * Sq + half * BLK
        for rdma in rdmas[c]:
            rdma.wait()
        out_ref[pl.ds(r0, BLK), :] = (
            out_ref[pl.ds(r0, BLK), :]
            + rp1_ref[c].astype(jnp.float32)
            + rp2_ref[c].astype(jnp.float32)
            + rpd_ref[c].astype(jnp.float32)
        )


def kernel(x, Wq, K_ext, V_ext, Wo):
    my_i = lax.axis_index("i")
    Kh = lax.dynamic_slice_in_dim(K_ext, my_i * HL, HL, axis=2)
    Vh = lax.dynamic_slice_in_dim(V_ext, my_i * HL, HL, axis=2)

    out = pl.pallas_call(
        _body,
        out_shape=jax.ShapeDtypeStruct((ROWS, D_MODEL), jnp.float32),
        in_specs=[pl.BlockSpec(memory_space=pltpu.VMEM)] * 5,
        out_specs=pl.BlockSpec(memory_space=pltpu.VMEM),
        scratch_shapes=[
            pltpu.VMEM((BLK, D_HEADS), jnp.bfloat16),
            pltpu.VMEM((NCHUNK, BLK, D_MODEL), jnp.bfloat16),
            pltpu.VMEM((NCHUNK, BLK, D_MODEL), jnp.bfloat16),
            pltpu.VMEM((NCHUNK, BLK, D_MODEL), jnp.bfloat16),
            pltpu.VMEM((NCHUNK, BLK, D_MODEL), jnp.bfloat16),
            pltpu.SemaphoreType.DMA((3, NCHUNK)),
            pltpu.SemaphoreType.DMA((3, NCHUNK)),
        ],
        compiler_params=pltpu.CompilerParams(collective_id=0),
    )(
        x.reshape(ROWS, D_MODEL),
        Wq,
        Kh.reshape(B * Skv, D_HEADS),
        Vh.reshape(B * Skv, D_HEADS),
        Wo,
    )
    return out.reshape(B, Sq, D_MODEL)
